# Initial kernel scaffold; baseline (speedup 1.0000x reference)
import jax
import jax.numpy as jnp
from jax.experimental import pallas as pl


def kernel(xyz, features, geo_w1, geo_b1, geo_g1, geo_be1, geo_w2, geo_b2, diff_w1, diff_b1, diff_g1, diff_be1, diff_w2, diff_b2, ep_w1, ep_b1, ep_w2, ep_b2, ref_w1, ref_b1, ref_g, ref_be):
    raise NotImplementedError("write your pallas kernel here")


# speedup vs baseline: 1.0154x; 1.0154x over previous
import jax
import jax.numpy as jnp
from jax.experimental import pallas as pl

K = 16
_BLK = 256


def _ln(x, g, b, eps=1e-5):
    mu = jnp.mean(x, axis=-1, keepdims=True)
    var = jnp.mean((x - mu) ** 2, axis=-1, keepdims=True)
    return (x - mu) / jnp.sqrt(var + eps) * g + b


def _knn(x, k):
    inner = -2.0 * jnp.matmul(x, jnp.swapaxes(x, 2, 1))
    xx = jnp.sum(x ** 2, axis=2, keepdims=True)
    pd = -xx - inner - jnp.swapaxes(xx, 2, 1)
    _, idx = jax.lax.top_k(pd, k)
    return idx


def _index_points(points, idx):
    return jax.vmap(lambda p, i: p[i])(points, idx)


def _tail_kernel(ef_ref, f_ref, epw1_ref, epb1_ref, epw2_ref, epb2_ref,
                 rw1_ref, rb1_ref, rg_ref, rbe_ref, refined_ref, prob_ref):
    ef = ef_ref[0]
    f = f_ref[0]
    e = jnp.maximum(jnp.dot(ef, epw1_ref[...],
                            preferred_element_type=jnp.float32) + epb1_ref[...], 0.0)
    logit = jnp.dot(e, epw2_ref[...], preferred_element_type=jnp.float32) + epb2_ref[...]
    prob = jax.nn.sigmoid(logit)
    enhanced = jnp.concatenate([f, ef * prob], axis=-1)
    r = jnp.dot(enhanced, rw1_ref[...], preferred_element_type=jnp.float32) + rb1_ref[...]
    mu = jnp.mean(r, axis=-1, keepdims=True)
    var = jnp.mean((r - mu) ** 2, axis=-1, keepdims=True)
    r = (r - mu) / jnp.sqrt(var + 1e-5) * rg_ref[...] + rbe_ref[...]
    r = jnp.maximum(r, 0.0)
    refined_ref[0] = r + f
    prob_ref[0] = prob


def kernel(xyz, features, geo_w1, geo_b1, geo_g1, geo_be1, geo_w2, geo_b2,
           diff_w1, diff_b1, diff_g1, diff_be1, diff_w2, diff_b2,
           ep_w1, ep_b1, ep_w2, ep_b2, ref_w1, ref_b1, ref_g, ref_be):
    B, N, C = features.shape
    idx = _knn(xyz, K)
    neighbors = _index_points(xyz, idx)
    rel_pos = neighbors - xyz[:, :, None, :]
    cov = jnp.matmul(jnp.swapaxes(rel_pos, -2, -1), rel_pos) / K
    evals, evecs = jnp.linalg.eigh(cov)
    normal = evecs[..., 0]
    curvature_dir = evecs[..., 2]
    mean_rel = rel_pos.mean(axis=2)
    mean_dist = jnp.linalg.norm(rel_pos, axis=-1).mean(axis=-1, keepdims=True)
    geo_feat = jnp.concatenate([normal, curvature_dir, mean_rel, mean_dist], axis=-1)
    h = geo_feat @ geo_w1 + geo_b1
    h = jax.nn.relu(_ln(h, geo_g1, geo_be1))
    geo_encoded = h @ geo_w2 + geo_b2
    neighbor_feats = _index_points(features, idx)
    feat_diff = jnp.abs(features[:, :, None, :] - neighbor_feats).mean(axis=2)
    d = feat_diff @ diff_w1 + diff_b1
    d = jax.nn.relu(_ln(d, diff_g1, diff_be1))
    diff_encoded = d @ diff_w2 + diff_b2
    edge_feat = jnp.concatenate([geo_encoded, diff_encoded], axis=-1)

    grid = (B, N // _BLK)
    refined, edge_prob = pl.pallas_call(
        _tail_kernel,
        grid=grid,
        in_specs=[
            pl.BlockSpec((1, _BLK, C), lambda b, i: (b, i, 0)),
            pl.BlockSpec((1, _BLK, C), lambda b, i: (b, i, 0)),
            pl.BlockSpec((128, 32), lambda b, i: (0, 0)),
            pl.BlockSpec((32,), lambda b, i: (0,)),
            pl.BlockSpec((32, 1), lambda b, i: (0, 0)),
            pl.BlockSpec((1,), lambda b, i: (0,)),
            pl.BlockSpec((256, 128), lambda b, i: (0, 0)),
            pl.BlockSpec((128,), lambda b, i: (0,)),
            pl.BlockSpec((128,), lambda b, i: (0,)),
            pl.BlockSpec((128,), lambda b, i: (0,)),
        ],
        out_specs=[
            pl.BlockSpec((1, _BLK, C), lambda b, i: (b, i, 0)),
            pl.BlockSpec((1, _BLK, 1), lambda b, i: (b, i, 0)),
        ],
        out_shape=[
            jax.ShapeDtypeStruct((B, N, C), jnp.float32),
            jax.ShapeDtypeStruct((B, N, 1), jnp.float32),
        ],
    )(edge_feat, features, ep_w1, ep_b1, ep_w2, ep_b2,
      ref_w1, ref_b1, ref_g, ref_be)
    return (refined, edge_prob)


# speedup vs baseline: 11.1945x; 11.0247x over previous
import functools

import jax
import jax.numpy as jnp
from jax.experimental import pallas as pl
from jax.experimental.pallas import tpu as pltpu

K = 16
N = 4096
_BLK = 512
_NEG = -3.0e38


def _knn_geo_kernel(xyz_blk_ref, xyz_all_ref,
                    gw1_ref, gb1_ref, gg1_ref, gbe1_ref, gw2_ref, gb2_ref,
                    idx_ref, geo_ref, pd_ref):
    b = pl.program_id(0)
    xyz_blk = xyz_blk_ref[0]
    xyz_all = xyz_all_ref[0]
    inner = -2.0 * jnp.dot(xyz_blk, xyz_all.T,
                           preferred_element_type=jnp.float32)
    xx_blk = jnp.sum(xyz_blk * xyz_blk, axis=-1, keepdims=True)
    xx_all = jnp.sum(xyz_all * xyz_all, axis=-1, keepdims=True)
    pd_ref[...] = (-xx_blk - inner) - xx_all.T

    iota = jax.lax.broadcasted_iota(jnp.int32, (_BLK, N), 1)
    col16 = jax.lax.broadcasted_iota(jnp.int32, (_BLK, K), 1)

    def body(k, carry):
        idxs, dsum, srel, srr = carry
        pd = pd_ref[...]
        m = jnp.max(pd, axis=1, keepdims=True)
        am = jnp.min(jnp.where(pd >= m, iota, N), axis=1, keepdims=True)
        idxs = jnp.where(col16 == k, am, idxs)
        hit = iota == am
        pd_ref[...] = jnp.where(hit, _NEG, pd)
        nbr = jnp.concatenate(
            [jnp.max(jnp.where(hit, xyz_all[:, c:c + 1].T, _NEG),
                     axis=1, keepdims=True) for c in range(3)], axis=-1)
        rel = nbr - xyz_blk
        dsum = dsum + jnp.sqrt(
            jnp.sum(rel * rel, axis=-1, keepdims=True))
        srel = srel + rel
        relb = rel.astype(jnp.bfloat16).astype(jnp.float32)
        rx = relb[:, 0:1]
        ry = relb[:, 1:2]
        rz = relb[:, 2:3]
        srr = srr + jnp.concatenate(
            [rx * rx, rx * ry, rx * rz, ry * ry, ry * rz, rz * rz], axis=-1)
        return idxs, dsum, srel, srr

    idxs0 = jnp.zeros((_BLK, K), jnp.int32)
    idxs, dsum, srel, srr = jax.lax.fori_loop(
        0, K, body, (idxs0, jnp.zeros((_BLK, 1)), jnp.zeros((_BLK, 3)),
                     jnp.zeros((_BLK, 6))))
    idx_ref[0] = idxs + b * N

    inv_k = 1.0 / K
    cidx = {(0, 0): 0, (0, 1): 1, (0, 2): 2, (1, 1): 3, (1, 2): 4, (2, 2): 5}
    a = [[srr[:, cidx[(min(i, j), max(i, j))]:cidx[(min(i, j), max(i, j))] + 1]
          * inv_k for j in range(3)] for i in range(3)]
    mean_rel = [srel[:, i:i + 1] for i in range(3)]
    mean_rel = [mr * inv_k for mr in mean_rel]
    mean_dist = dsum * inv_k

    v = [[jnp.full((_BLK, 1), 1.0 if i == j else 0.0) for j in range(3)]
         for i in range(3)]
    for _ in range(15):
        for (p, q) in ((0, 2), (1, 2), (0, 1)):
            apq = a[p][q]
            app = a[p][p]
            aqq = a[q][q]
            safe = jnp.where(apq == 0.0, 1.0, apq)
            tau = (aqq - app) / (2.0 * safe)
            sg = jnp.where(tau >= 0.0, 1.0, -1.0)
            t = sg / (jnp.abs(tau) + jnp.sqrt(1.0 + tau * tau))
            t = jnp.where(apq == 0.0, 0.0, t)
            cc = 1.0 / jnp.sqrt(1.0 + t * t)
            ss = t * cc
            r = 3 - p - q
            new_app = app - t * apq
            new_aqq = aqq + t * apq
            arp = a[r][p]
            arq = a[r][q]
            new_arp = cc * arp - ss * arq
            new_arq = ss * arp + cc * arq
            a[p][p] = new_app
            a[q][q] = new_aqq
            a[p][q] = jnp.zeros_like(apq)
            a[q][p] = a[p][q]
            a[r][p] = new_arp
            a[p][r] = new_arp
            a[r][q] = new_arq
            a[q][r] = new_arq
            for i in range(3):
                vip = v[i][p]
                viq = v[i][q]
                v[i][p] = cc * vip - ss * viq
                v[i][q] = ss * vip + cc * viq

    d0, d1, d2 = a[0][0], a[1][1], a[2][2]
    r0 = (d1 < d0).astype(jnp.int32) + (d2 < d0).astype(jnp.int32)
    r1 = (d0 <= d1).astype(jnp.int32) + (d2 < d1).astype(jnp.int32)
    r2 = (d0 <= d2).astype(jnp.int32) + (d1 <= d2).astype(jnp.int32)

    def pick(slot):
        return [jnp.where(r0 == slot, v[i][0],
                          jnp.where(r1 == slot, v[i][1], v[i][2]))
                for i in range(3)]

    normal = pick(0)
    curv = pick(2)
    geo_feat = jnp.concatenate(
        normal + curv + mean_rel + [mean_dist, jnp.zeros((_BLK, 6))], axis=-1)

    h = jnp.dot(geo_feat, gw1_ref[...],
                preferred_element_type=jnp.float32) + gb1_ref[...]
    mu = jnp.mean(h, axis=-1, keepdims=True)
    var = jnp.mean((h - mu) ** 2, axis=-1, keepdims=True)
    h = (h - mu) / jnp.sqrt(var + 1e-5) * gg1_ref[...] + gbe1_ref[...]
    h = jnp.maximum(h, 0.0)
    geo_ref[0] = jnp.dot(h, gw2_ref[...],
                         preferred_element_type=jnp.float32) + gb2_ref[...]


def _tail_kernel(fd_ref, f_ref, dw1_ref, db1_ref, dg1_ref, dbe1_ref,
                 dw2_ref, db2_ref, geo_ref,
                 epw1_ref, epb1_ref, epw2_ref, epb2_ref,
                 rw1_ref, rb1_ref, rg_ref, rbe_ref, refined_ref, prob_ref):
    fd = fd_ref[0]
    f = f_ref[0]
    d = jnp.dot(fd, dw1_ref[...], preferred_element_type=jnp.float32) + db1_ref[...]
    mu = jnp.mean(d, axis=-1, keepdims=True)
    var = jnp.mean((d - mu) ** 2, axis=-1, keepdims=True)
    d = (d - mu) / jnp.sqrt(var + 1e-5) * dg1_ref[...] + dbe1_ref[...]
    d = jnp.maximum(d, 0.0)
    diff_enc = jnp.dot(d, dw2_ref[...],
                       preferred_element_type=jnp.float32) + db2_ref[...]
    ef = jnp.concatenate([geo_ref[0], diff_enc], axis=-1)
    e = jnp.maximum(jnp.dot(ef, epw1_ref[...],
                            preferred_element_type=jnp.float32) + epb1_ref[...], 0.0)
    logit = jnp.dot(e, epw2_ref[...],
                    preferred_element_type=jnp.float32) + epb2_ref[...]
    prob = jax.nn.sigmoid(logit)
    enhanced = jnp.concatenate([f, ef * prob], axis=-1)
    r = jnp.dot(enhanced, rw1_ref[...],
                preferred_element_type=jnp.float32) + rb1_ref[...]
    mu = jnp.mean(r, axis=-1, keepdims=True)
    var = jnp.mean((r - mu) ** 2, axis=-1, keepdims=True)
    r = (r - mu) / jnp.sqrt(var + 1e-5) * rg_ref[...] + rbe_ref[...]
    r = jnp.maximum(r, 0.0)
    refined_ref[0] = r + f
    prob_ref[0] = prob


def kernel(xyz, features, geo_w1, geo_b1, geo_g1, geo_be1, geo_w2, geo_b2,
           diff_w1, diff_b1, diff_g1, diff_be1, diff_w2, diff_b2,
           ep_w1, ep_b1, ep_w2, ep_b2, ref_w1, ref_b1, ref_g, ref_be):
    B, n, C = features.shape
    grid = (B, n // _BLK)
    flat_idx, geo_enc = pl.pallas_call(
        _knn_geo_kernel,
        grid=grid,
        in_specs=[
            pl.BlockSpec((1, _BLK, 3), lambda b, i: (b, i, 0)),
            pl.BlockSpec((1, N, 3), lambda b, i: (b, 0, 0)),
            pl.BlockSpec((16, 64), lambda b, i: (0, 0)),
            pl.BlockSpec((64,), lambda b, i: (0,)),
            pl.BlockSpec((64,), lambda b, i: (0,)),
            pl.BlockSpec((64,), lambda b, i: (0,)),
            pl.BlockSpec((64, 64), lambda b, i: (0, 0)),
            pl.BlockSpec((64,), lambda b, i: (0,)),
        ],
        out_specs=[
            pl.BlockSpec((1, _BLK, K), lambda b, i: (b * (n // _BLK) + i, 0, 0)),
            pl.BlockSpec((1, _BLK, 64), lambda b, i: (b * (n // _BLK) + i, 0, 0)),
        ],
        out_shape=[
            jax.ShapeDtypeStruct((B * n // _BLK, _BLK, K), jnp.int32),
            jax.ShapeDtypeStruct((B * n // _BLK, _BLK, 64), jnp.float32),
        ],
        scratch_shapes=[pltpu.VMEM((_BLK, N), jnp.float32)],
    )(xyz, xyz, _pad10(geo_w1), geo_b1, geo_g1, geo_be1, geo_w2, geo_b2)
    flat_idx = flat_idx.reshape(B * n, K)
    geo_enc = geo_enc.reshape(B, n, 64)

    feats_flat = features.reshape(B * n, C)
    nbr = feats_flat[flat_idx]
    feat_diff = jnp.abs(feats_flat[:, None, :] - nbr).mean(axis=1)
    feat_diff = feat_diff.reshape(B, n, C)

    refined, edge_prob = pl.pallas_call(
        _tail_kernel,
        grid=grid,
        in_specs=[
            pl.BlockSpec((1, _BLK, C), lambda b, i: (b, i, 0)),
            pl.BlockSpec((1, _BLK, C), lambda b, i: (b, i, 0)),
            pl.BlockSpec((128, 64), lambda b, i: (0, 0)),
            pl.BlockSpec((64,), lambda b, i: (0,)),
            pl.BlockSpec((64,), lambda b, i: (0,)),
            pl.BlockSpec((64,), lambda b, i: (0,)),
            pl.BlockSpec((64, 64), lambda b, i: (0, 0)),
            pl.BlockSpec((64,), lambda b, i: (0,)),
            pl.BlockSpec((1, _BLK, 64), lambda b, i: (b, i, 0)),
            pl.BlockSpec((128, 32), lambda b, i: (0, 0)),
            pl.BlockSpec((32,), lambda b, i: (0,)),
            pl.BlockSpec((32, 1), lambda b, i: (0, 0)),
            pl.BlockSpec((1,), lambda b, i: (0,)),
            pl.BlockSpec((256, 128), lambda b, i: (0, 0)),
            pl.BlockSpec((128,), lambda b, i: (0,)),
            pl.BlockSpec((128,), lambda b, i: (0,)),
            pl.BlockSpec((128,), lambda b, i: (0,)),
        ],
        out_specs=[
            pl.BlockSpec((1, _BLK, C), lambda b, i: (b, i, 0)),
            pl.BlockSpec((1, _BLK, 1), lambda b, i: (b, i, 0)),
        ],
        out_shape=[
            jax.ShapeDtypeStruct((B, n, C), jnp.float32),
            jax.ShapeDtypeStruct((B, n, 1), jnp.float32),
        ],
    )(feat_diff, features, diff_w1, diff_b1, diff_g1, diff_be1,
      diff_w2, diff_b2, geo_enc, ep_w1, ep_b1, ep_w2, ep_b2,
      ref_w1, ref_b1, ref_g, ref_be)
    return (refined, edge_prob)


def _pad10(w):
    return jnp.pad(w, ((0, 6), (0, 0)))


# speedup vs baseline: 12.6241x; 1.1277x over previous
import functools

import jax
import jax.numpy as jnp
from jax import lax
from jax.experimental import pallas as pl
from jax.experimental.pallas import tpu as pltpu
from jax.experimental.pallas import tpu_sc as plsc

K = 16
N = 4096
_BLK = 512
_NEG = -3.0e38


def _knn_geo_kernel(xyz_blk_ref, xyz_all_ref,
                    gw1_ref, gb1_ref, gg1_ref, gbe1_ref, gw2_ref, gb2_ref,
                    idx_ref, geo_ref, pd_ref):
    b = pl.program_id(0)
    xyz_blk = xyz_blk_ref[0]
    xyz_all = xyz_all_ref[0]
    inner = -2.0 * jnp.dot(xyz_blk, xyz_all.T,
                           preferred_element_type=jnp.float32)
    xx_blk = jnp.sum(xyz_blk * xyz_blk, axis=-1, keepdims=True)
    xx_all = jnp.sum(xyz_all * xyz_all, axis=-1, keepdims=True)
    pd_ref[...] = (-xx_blk - inner) - xx_all.T

    iota = jax.lax.broadcasted_iota(jnp.int32, (_BLK, N), 1)
    col16 = jax.lax.broadcasted_iota(jnp.int32, (_BLK, K), 1)

    def body(k, carry):
        idxs, dsum, srel, srr = carry
        pd = pd_ref[...]
        m = jnp.max(pd, axis=1, keepdims=True)
        am = jnp.min(jnp.where(pd >= m, iota, N), axis=1, keepdims=True)
        idxs = jnp.where(col16 == k, am, idxs)
        hit = iota == am
        pd_ref[...] = jnp.where(hit, _NEG, pd)
        nbr = jnp.concatenate(
            [jnp.max(jnp.where(hit, xyz_all[:, c:c + 1].T, _NEG),
                     axis=1, keepdims=True) for c in range(3)], axis=-1)
        rel = nbr - xyz_blk
        dsum = dsum + jnp.sqrt(
            jnp.sum(rel * rel, axis=-1, keepdims=True))
        srel = srel + rel
        relb = rel.astype(jnp.bfloat16).astype(jnp.float32)
        rx = relb[:, 0:1]
        ry = relb[:, 1:2]
        rz = relb[:, 2:3]
        srr = srr + jnp.concatenate(
            [rx * rx, rx * ry, rx * rz, ry * ry, ry * rz, rz * rz], axis=-1)
        return idxs, dsum, srel, srr

    idxs0 = jnp.zeros((_BLK, K), jnp.int32)
    idxs, dsum, srel, srr = jax.lax.fori_loop(
        0, K, body, (idxs0, jnp.zeros((_BLK, 1)), jnp.zeros((_BLK, 3)),
                     jnp.zeros((_BLK, 6))))
    idx_ref[0] = idxs + b * N

    inv_k = 1.0 / K
    cidx = {(0, 0): 0, (0, 1): 1, (0, 2): 2, (1, 1): 3, (1, 2): 4, (2, 2): 5}
    a = [[srr[:, cidx[(min(i, j), max(i, j))]:cidx[(min(i, j), max(i, j))] + 1]
          * inv_k for j in range(3)] for i in range(3)]
    mean_rel = [srel[:, i:i + 1] for i in range(3)]
    mean_rel = [mr * inv_k for mr in mean_rel]
    mean_dist = dsum * inv_k

    v = [[jnp.full((_BLK, 1), 1.0 if i == j else 0.0) for j in range(3)]
         for i in range(3)]
    for _ in range(15):
        for (p, q) in ((0, 2), (1, 2), (0, 1)):
            apq = a[p][q]
            app = a[p][p]
            aqq = a[q][q]
            safe = jnp.where(apq == 0.0, 1.0, apq)
            tau = (aqq - app) / (2.0 * safe)
            sg = jnp.where(tau >= 0.0, 1.0, -1.0)
            t = sg / (jnp.abs(tau) + jnp.sqrt(1.0 + tau * tau))
            t = jnp.where(apq == 0.0, 0.0, t)
            cc = 1.0 / jnp.sqrt(1.0 + t * t)
            ss = t * cc
            r = 3 - p - q
            new_app = app - t * apq
            new_aqq = aqq + t * apq
            arp = a[r][p]
            arq = a[r][q]
            new_arp = cc * arp - ss * arq
            new_arq = ss * arp + cc * arq
            a[p][p] = new_app
            a[q][q] = new_aqq
            a[p][q] = jnp.zeros_like(apq)
            a[q][p] = a[p][q]
            a[r][p] = new_arp
            a[p][r] = new_arp
            a[r][q] = new_arq
            a[q][r] = new_arq
            for i in range(3):
                vip = v[i][p]
                viq = v[i][q]
                v[i][p] = cc * vip - ss * viq
                v[i][q] = ss * vip + cc * viq

    d0, d1, d2 = a[0][0], a[1][1], a[2][2]
    r0 = (d1 < d0).astype(jnp.int32) + (d2 < d0).astype(jnp.int32)
    r1 = (d0 <= d1).astype(jnp.int32) + (d2 < d1).astype(jnp.int32)
    r2 = (d0 <= d2).astype(jnp.int32) + (d1 <= d2).astype(jnp.int32)

    def pick(slot):
        return [jnp.where(r0 == slot, v[i][0],
                          jnp.where(r1 == slot, v[i][1], v[i][2]))
                for i in range(3)]

    normal = pick(0)
    curv = pick(2)
    geo_feat = jnp.concatenate(
        normal + curv + mean_rel + [mean_dist, jnp.zeros((_BLK, 6))], axis=-1)

    h = jnp.dot(geo_feat, gw1_ref[...],
                preferred_element_type=jnp.float32) + gb1_ref[...]
    mu = jnp.mean(h, axis=-1, keepdims=True)
    var = jnp.mean((h - mu) ** 2, axis=-1, keepdims=True)
    h = (h - mu) / jnp.sqrt(var + 1e-5) * gg1_ref[...] + gbe1_ref[...]
    h = jnp.maximum(h, 0.0)
    geo_ref[0] = jnp.dot(h, gw2_ref[...],
                         preferred_element_type=jnp.float32) + gb2_ref[...]


_NW = 32
_PPW = (2 * N) // _NW
_CHP = 16


def _sc_diff_kernel(feats_hbm, idxflat_hbm, out_hbm, idx_v, rows_v, own_v,
                    out_v, sem):
    wid = lax.axis_index("s") * 2 + lax.axis_index("c")

    def chunk_body(g, _):
        base_pt = wid * _PPW + g * _CHP
        pltpu.sync_copy(idxflat_hbm.at[pl.ds(base_pt * K, _CHP * K)], idx_v)
        pltpu.async_copy(feats_hbm.at[idx_v], rows_v, sem).wait()
        pltpu.sync_copy(feats_hbm.at[pl.ds(base_pt, _CHP), :], own_v)

        def point_body(i, _):
            for j in range(8):
                own_j = own_v[i, pl.ds(16 * j, 16)]
                acc = jnp.abs(own_j - rows_v[i * K, pl.ds(16 * j, 16)])
                for k in range(1, K):
                    acc = acc + jnp.abs(
                        own_j - rows_v[i * K + k, pl.ds(16 * j, 16)])
                out_v[i, pl.ds(16 * j, 16)] = acc * (1.0 / K)
            return 0

        lax.fori_loop(0, _CHP, point_body, 0)
        pltpu.sync_copy(out_v, out_hbm.at[pl.ds(base_pt, _CHP), :])
        return 0

    lax.fori_loop(0, _PPW // _CHP, chunk_body, 0)


def _sc_feat_diff(feats_flat, idx_flat):
    mesh = plsc.VectorSubcoreMesh(core_axis_name="c", subcore_axis_name="s")
    run = pl.kernel(
        _sc_diff_kernel,
        mesh=mesh,
        out_type=jax.ShapeDtypeStruct(feats_flat.shape, jnp.float32),
        scratch_types=[
            pltpu.VMEM((_CHP * K,), jnp.int32),
            pltpu.VMEM((_CHP * K, 128), jnp.float32),
            pltpu.VMEM((_CHP, 128), jnp.float32),
            pltpu.VMEM((_CHP, 128), jnp.float32),
            pltpu.SemaphoreType.DMA,
        ],
    )
    return run(feats_flat, idx_flat)


def _tail_kernel(fd_ref, f_ref, dw1_ref, db1_ref, dg1_ref, dbe1_ref,
                 dw2_ref, db2_ref, geo_ref,
                 epw1_ref, epb1_ref, epw2_ref, epb2_ref,
                 rw1_ref, rb1_ref, rg_ref, rbe_ref, refined_ref, prob_ref):
    fd = fd_ref[0]
    f = f_ref[0]
    d = jnp.dot(fd, dw1_ref[...], preferred_element_type=jnp.float32) + db1_ref[...]
    mu = jnp.mean(d, axis=-1, keepdims=True)
    var = jnp.mean((d - mu) ** 2, axis=-1, keepdims=True)
    d = (d - mu) / jnp.sqrt(var + 1e-5) * dg1_ref[...] + dbe1_ref[...]
    d = jnp.maximum(d, 0.0)
    diff_enc = jnp.dot(d, dw2_ref[...],
                       preferred_element_type=jnp.float32) + db2_ref[...]
    ef = jnp.concatenate([geo_ref[0], diff_enc], axis=-1)
    e = jnp.maximum(jnp.dot(ef, epw1_ref[...],
                            preferred_element_type=jnp.float32) + epb1_ref[...], 0.0)
    logit = jnp.dot(e, epw2_ref[...],
                    preferred_element_type=jnp.float32) + epb2_ref[...]
    prob = jax.nn.sigmoid(logit)
    enhanced = jnp.concatenate([f, ef * prob], axis=-1)
    r = jnp.dot(enhanced, rw1_ref[...],
                preferred_element_type=jnp.float32) + rb1_ref[...]
    mu = jnp.mean(r, axis=-1, keepdims=True)
    var = jnp.mean((r - mu) ** 2, axis=-1, keepdims=True)
    r = (r - mu) / jnp.sqrt(var + 1e-5) * rg_ref[...] + rbe_ref[...]
    r = jnp.maximum(r, 0.0)
    refined_ref[0] = r + f
    prob_ref[0] = prob


def kernel(xyz, features, geo_w1, geo_b1, geo_g1, geo_be1, geo_w2, geo_b2,
           diff_w1, diff_b1, diff_g1, diff_be1, diff_w2, diff_b2,
           ep_w1, ep_b1, ep_w2, ep_b2, ref_w1, ref_b1, ref_g, ref_be):
    B, n, C = features.shape
    grid = (B, n // _BLK)
    flat_idx, geo_enc = pl.pallas_call(
        _knn_geo_kernel,
        grid=grid,
        in_specs=[
            pl.BlockSpec((1, _BLK, 3), lambda b, i: (b, i, 0)),
            pl.BlockSpec((1, N, 3), lambda b, i: (b, 0, 0)),
            pl.BlockSpec((16, 64), lambda b, i: (0, 0)),
            pl.BlockSpec((64,), lambda b, i: (0,)),
            pl.BlockSpec((64,), lambda b, i: (0,)),
            pl.BlockSpec((64,), lambda b, i: (0,)),
            pl.BlockSpec((64, 64), lambda b, i: (0, 0)),
            pl.BlockSpec((64,), lambda b, i: (0,)),
        ],
        out_specs=[
            pl.BlockSpec((1, _BLK, K), lambda b, i: (b * (n // _BLK) + i, 0, 0)),
            pl.BlockSpec((1, _BLK, 64), lambda b, i: (b * (n // _BLK) + i, 0, 0)),
        ],
        out_shape=[
            jax.ShapeDtypeStruct((B * n // _BLK, _BLK, K), jnp.int32),
            jax.ShapeDtypeStruct((B * n // _BLK, _BLK, 64), jnp.float32),
        ],
        scratch_shapes=[pltpu.VMEM((_BLK, N), jnp.float32)],
    )(xyz, xyz, _pad10(geo_w1), geo_b1, geo_g1, geo_be1, geo_w2, geo_b2)
    flat_idx = flat_idx.reshape(B * n, K)
    geo_enc = geo_enc.reshape(B, n, 64)

    feats_flat = features.reshape(B * n, C)
    feat_diff = _sc_feat_diff(feats_flat, flat_idx.reshape(B * n * K))
    feat_diff = feat_diff.reshape(B, n, C)

    refined, edge_prob = pl.pallas_call(
        _tail_kernel,
        grid=grid,
        in_specs=[
            pl.BlockSpec((1, _BLK, C), lambda b, i: (b, i, 0)),
            pl.BlockSpec((1, _BLK, C), lambda b, i: (b, i, 0)),
            pl.BlockSpec((128, 64), lambda b, i: (0, 0)),
            pl.BlockSpec((64,), lambda b, i: (0,)),
            pl.BlockSpec((64,), lambda b, i: (0,)),
            pl.BlockSpec((64,), lambda b, i: (0,)),
            pl.BlockSpec((64, 64), lambda b, i: (0, 0)),
            pl.BlockSpec((64,), lambda b, i: (0,)),
            pl.BlockSpec((1, _BLK, 64), lambda b, i: (b, i, 0)),
            pl.BlockSpec((128, 32), lambda b, i: (0, 0)),
            pl.BlockSpec((32,), lambda b, i: (0,)),
            pl.BlockSpec((32, 1), lambda b, i: (0, 0)),
            pl.BlockSpec((1,), lambda b, i: (0,)),
            pl.BlockSpec((256, 128), lambda b, i: (0, 0)),
            pl.BlockSpec((128,), lambda b, i: (0,)),
            pl.BlockSpec((128,), lambda b, i: (0,)),
            pl.BlockSpec((128,), lambda b, i: (0,)),
        ],
        out_specs=[
            pl.BlockSpec((1, _BLK, C), lambda b, i: (b, i, 0)),
            pl.BlockSpec((1, _BLK, 1), lambda b, i: (b, i, 0)),
        ],
        out_shape=[
            jax.ShapeDtypeStruct((B, n, C), jnp.float32),
            jax.ShapeDtypeStruct((B, n, 1), jnp.float32),
        ],
    )(feat_diff, features, diff_w1, diff_b1, diff_g1, diff_be1,
      diff_w2, diff_b2, geo_enc, ep_w1, ep_b1, ep_w2, ep_b2,
      ref_w1, ref_b1, ref_g, ref_be)
    return (refined, edge_prob)


def _pad10(w):
    return jnp.pad(w, ((0, 6), (0, 0)))


# speedup vs baseline: 20.7166x; 1.6410x over previous
import functools

import jax
import jax.numpy as jnp
from jax import lax
from jax.experimental import pallas as pl
from jax.experimental.pallas import tpu as pltpu
from jax.experimental.pallas import tpu_sc as plsc

K = 16
N = 4096
_BLK = 512
_NEG = -3.0e38

_NW = 32
_PPW = (2 * N) // _NW
_CHP = 16


def _knn_kernel(xyz_blk_ref, xyz_all_ref, idx_ref, pd_ref):
    b = pl.program_id(0)
    xyz_blk = xyz_blk_ref[0]
    xyz_all = xyz_all_ref[0]
    inner = -2.0 * jnp.dot(xyz_blk, xyz_all.T,
                           preferred_element_type=jnp.float32)
    xx_blk = jnp.sum(xyz_blk * xyz_blk, axis=-1, keepdims=True)
    xx_all = jnp.sum(xyz_all * xyz_all, axis=-1, keepdims=True)
    pd_ref[...] = (-xx_blk - inner) - xx_all.T

    iota = jax.lax.broadcasted_iota(jnp.int32, (_BLK, N), 1)
    col16 = jax.lax.broadcasted_iota(jnp.int32, (_BLK, K), 1)

    def body(k, idxs):
        pd = pd_ref[...]
        m = jnp.max(pd, axis=1, keepdims=True)
        am = jnp.min(jnp.where(pd >= m, iota, N), axis=1, keepdims=True)
        pd_ref[...] = jnp.where(iota == am, _NEG, pd)
        return jnp.where(col16 == k, am, idxs)

    idxs = jax.lax.fori_loop(0, K, body, jnp.zeros((_BLK, K), jnp.int32))
    idx_ref[0] = idxs + b * N


def _sc_gather_kernel(feats_hbm, idxflat_hbm, xyzp_hbm,
                      fd_hbm, nbrxyz_hbm,
                      idx_v, rows_v, own_v, out_v, xyzrows_v, xyzpack_v,
                      sem, sem2):
    wid = lax.axis_index("s") * 2 + lax.axis_index("c")

    def chunk_body(g, _):
        base_pt = wid * _PPW + g * _CHP
        pltpu.sync_copy(idxflat_hbm.at[pl.ds(base_pt * K, _CHP * K)], idx_v)
        cp_f = pltpu.async_copy(feats_hbm.at[idx_v], rows_v, sem)
        cp_x = pltpu.async_copy(xyzp_hbm.at[idx_v], xyzrows_v, sem2)
        pltpu.sync_copy(feats_hbm.at[pl.ds(base_pt, _CHP), :], own_v)
        cp_x.wait()

        def pack_body(r, _):
            xyzpack_v[r, :] = xyzrows_v[r, pl.ds(0, 16)]
            return 0

        lax.fori_loop(0, _CHP * K, pack_body, 0)
        pltpu.sync_copy(xyzpack_v,
                        nbrxyz_hbm.at[pl.ds(base_pt * K, _CHP * K), :])
        cp_f.wait()

        def point_body(i, _):
            for j in range(8):
                own_j = own_v[i, pl.ds(16 * j, 16)]
                acc = jnp.abs(own_j - rows_v[i * K, pl.ds(16 * j, 16)])
                for k in range(1, K):
                    acc = acc + jnp.abs(
                        own_j - rows_v[i * K + k, pl.ds(16 * j, 16)])
                out_v[i, pl.ds(16 * j, 16)] = acc * (1.0 / K)
            return 0

        lax.fori_loop(0, _CHP, point_body, 0)
        pltpu.sync_copy(out_v, fd_hbm.at[pl.ds(base_pt, _CHP), :])
        return 0

    lax.fori_loop(0, _PPW // _CHP, chunk_body, 0)


def _sc_gather(feats_flat, idx_flat, xyz_pad):
    mesh = plsc.VectorSubcoreMesh(core_axis_name="c", subcore_axis_name="s")
    run = pl.kernel(
        _sc_gather_kernel,
        mesh=mesh,
        out_type=[
            jax.ShapeDtypeStruct(feats_flat.shape, jnp.float32),
            jax.ShapeDtypeStruct((feats_flat.shape[0] * K, 16), jnp.float32),
        ],
        scratch_types=[
            pltpu.VMEM((_CHP * K,), jnp.int32),
            pltpu.VMEM((_CHP * K, 128), jnp.float32),
            pltpu.VMEM((_CHP, 128), jnp.float32),
            pltpu.VMEM((_CHP, 128), jnp.float32),
            pltpu.VMEM((_CHP * K, 128), jnp.float32),
            pltpu.VMEM((_CHP * K, 16), jnp.float32),
            pltpu.SemaphoreType.DMA,
            pltpu.SemaphoreType.DMA,
        ],
    )
    return run(feats_flat, idx_flat, xyz_pad)


def _geo_tail_kernel(nbr_ref, xyz_blk_ref, fd_ref, f_ref,
                     gw1_ref, gb1_ref, gg1_ref, gbe1_ref, gw2_ref, gb2_ref,
                     dw1_ref, db1_ref, dg1_ref, dbe1_ref, dw2_ref, db2_ref,
                     epw1_ref, epb1_ref, epw2_ref, epb2_ref,
                     rw1_ref, rb1_ref, rg_ref, rbe_ref,
                     refined_ref, prob_ref):
    nb = nbr_ref[0]
    xyz_blk = xyz_blk_ref[0]
    c = [xyz_blk[:, i:i + 1] for i in range(3)]

    dsum = jnp.zeros((_BLK, 1))
    srel = [jnp.zeros((_BLK, 1)) for _ in range(3)]
    srr = [jnp.zeros((_BLK, 1)) for _ in range(6)]
    for k in range(K):
        rel = [nb[:, 16 * k + i:16 * k + i + 1] - c[i] for i in range(3)]
        dsum = dsum + jnp.sqrt(rel[0] * rel[0] + rel[1] * rel[1]
                               + rel[2] * rel[2])
        relb = [r.astype(jnp.bfloat16).astype(jnp.float32) for r in rel]
        srel = [srel[i] + rel[i] for i in range(3)]
        prods = [relb[0] * relb[0], relb[0] * relb[1], relb[0] * relb[2],
                 relb[1] * relb[1], relb[1] * relb[2], relb[2] * relb[2]]
        srr = [srr[i] + prods[i] for i in range(6)]

    inv_k = 1.0 / K
    cidx = {(0, 0): 0, (0, 1): 1, (0, 2): 2, (1, 1): 3, (1, 2): 4, (2, 2): 5}
    a = [[srr[cidx[(min(i, j), max(i, j))]] * inv_k for j in range(3)]
         for i in range(3)]
    mean_rel = [s * inv_k for s in srel]
    mean_dist = dsum * inv_k

    v = [[jnp.full((_BLK, 1), 1.0 if i == j else 0.0) for j in range(3)]
         for i in range(3)]
    for _ in range(15):
        for (p, q) in ((0, 2), (1, 2), (0, 1)):
            apq = a[p][q]
            app = a[p][p]
            aqq = a[q][q]
            safe = jnp.where(apq == 0.0, 1.0, apq)
            tau = (aqq - app) / (2.0 * safe)
            sg = jnp.where(tau >= 0.0, 1.0, -1.0)
            t = sg / (jnp.abs(tau) + jnp.sqrt(1.0 + tau * tau))
            t = jnp.where(apq == 0.0, 0.0, t)
            cc = 1.0 / jnp.sqrt(1.0 + t * t)
            ss = t * cc
            r = 3 - p - q
            new_app = app - t * apq
            new_aqq = aqq + t * apq
            arp = a[r][p]
            arq = a[r][q]
            new_arp = cc * arp - ss * arq
            new_arq = ss * arp + cc * arq
            a[p][p] = new_app
            a[q][q] = new_aqq
            a[p][q] = jnp.zeros_like(apq)
            a[q][p] = a[p][q]
            a[r][p] = new_arp
            a[p][r] = new_arp
            a[r][q] = new_arq
            a[q][r] = new_arq
            for i in range(3):
                vip = v[i][p]
                viq = v[i][q]
                v[i][p] = cc * vip - ss * viq
                v[i][q] = ss * vip + cc * viq

    d0, d1, d2 = a[0][0], a[1][1], a[2][2]
    r0 = (d1 < d0).astype(jnp.int32) + (d2 < d0).astype(jnp.int32)
    r1 = (d0 <= d1).astype(jnp.int32) + (d2 < d1).astype(jnp.int32)
    r2 = (d0 <= d2).astype(jnp.int32) + (d1 <= d2).astype(jnp.int32)

    def pick(slot):
        return [jnp.where(r0 == slot, v[i][0],
                          jnp.where(r1 == slot, v[i][1], v[i][2]))
                for i in range(3)]

    normal = pick(0)
    curv = pick(2)
    geo_feat = jnp.concatenate(
        normal + curv + mean_rel + [mean_dist, jnp.zeros((_BLK, 6))], axis=-1)

    h = jnp.dot(geo_feat, gw1_ref[...],
                preferred_element_type=jnp.float32) + gb1_ref[...]
    mu = jnp.mean(h, axis=-1, keepdims=True)
    var = jnp.mean((h - mu) ** 2, axis=-1, keepdims=True)
    h = (h - mu) / jnp.sqrt(var + 1e-5) * gg1_ref[...] + gbe1_ref[...]
    h = jnp.maximum(h, 0.0)
    geo_enc = jnp.dot(h, gw2_ref[...],
                      preferred_element_type=jnp.float32) + gb2_ref[...]

    fd = fd_ref[0]
    f = f_ref[0]
    d = jnp.dot(fd, dw1_ref[...], preferred_element_type=jnp.float32) + db1_ref[...]
    mu = jnp.mean(d, axis=-1, keepdims=True)
    var = jnp.mean((d - mu) ** 2, axis=-1, keepdims=True)
    d = (d - mu) / jnp.sqrt(var + 1e-5) * dg1_ref[...] + dbe1_ref[...]
    d = jnp.maximum(d, 0.0)
    diff_enc = jnp.dot(d, dw2_ref[...],
                       preferred_element_type=jnp.float32) + db2_ref[...]
    ef = jnp.concatenate([geo_enc, diff_enc], axis=-1)
    e = jnp.maximum(jnp.dot(ef, epw1_ref[...],
                            preferred_element_type=jnp.float32) + epb1_ref[...], 0.0)
    logit = jnp.dot(e, epw2_ref[...],
                    preferred_element_type=jnp.float32) + epb2_ref[...]
    prob = jax.nn.sigmoid(logit)
    enhanced = jnp.concatenate([f, ef * prob], axis=-1)
    r = jnp.dot(enhanced, rw1_ref[...],
                preferred_element_type=jnp.float32) + rb1_ref[...]
    mu = jnp.mean(r, axis=-1, keepdims=True)
    var = jnp.mean((r - mu) ** 2, axis=-1, keepdims=True)
    r = (r - mu) / jnp.sqrt(var + 1e-5) * rg_ref[...] + rbe_ref[...]
    r = jnp.maximum(r, 0.0)
    refined_ref[0] = r + f
    prob_ref[0] = prob


def kernel(xyz, features, geo_w1, geo_b1, geo_g1, geo_be1, geo_w2, geo_b2,
           diff_w1, diff_b1, diff_g1, diff_be1, diff_w2, diff_b2,
           ep_w1, ep_b1, ep_w2, ep_b2, ref_w1, ref_b1, ref_g, ref_be):
    B, n, C = features.shape
    grid = (B, n // _BLK)
    nblk = B * n // _BLK
    flat_idx = pl.pallas_call(
        _knn_kernel,
        grid=grid,
        in_specs=[
            pl.BlockSpec((1, _BLK, 3), lambda b, i: (b, i, 0)),
            pl.BlockSpec((1, N, 3), lambda b, i: (b, 0, 0)),
        ],
        out_specs=pl.BlockSpec((1, _BLK, K),
                               lambda b, i: (b * (n // _BLK) + i, 0, 0)),
        out_shape=jax.ShapeDtypeStruct((nblk, _BLK, K), jnp.int32),
        scratch_shapes=[pltpu.VMEM((_BLK, N), jnp.float32)],
    )(xyz, xyz)

    feats_flat = features.reshape(B * n, C)
    xyz_pad = jnp.pad(xyz.reshape(B * n, 3), ((0, 0), (0, 125)))
    feat_diff, nbrxyz = _sc_gather(feats_flat, flat_idx.reshape(B * n * K),
                                   xyz_pad)
    feat_diff = feat_diff.reshape(B, n, C)
    nbrxyz = nbrxyz.reshape(B, n, K * 16)

    refined, edge_prob = pl.pallas_call(
        _geo_tail_kernel,
        grid=grid,
        in_specs=[
            pl.BlockSpec((1, _BLK, K * 16), lambda b, i: (b, i, 0)),
            pl.BlockSpec((1, _BLK, 3), lambda b, i: (b, i, 0)),
            pl.BlockSpec((1, _BLK, C), lambda b, i: (b, i, 0)),
            pl.BlockSpec((1, _BLK, C), lambda b, i: (b, i, 0)),
            pl.BlockSpec((16, 64), lambda b, i: (0, 0)),
            pl.BlockSpec((64,), lambda b, i: (0,)),
            pl.BlockSpec((64,), lambda b, i: (0,)),
            pl.BlockSpec((64,), lambda b, i: (0,)),
            pl.BlockSpec((64, 64), lambda b, i: (0, 0)),
            pl.BlockSpec((64,), lambda b, i: (0,)),
            pl.BlockSpec((128, 64), lambda b, i: (0, 0)),
            pl.BlockSpec((64,), lambda b, i: (0,)),
            pl.BlockSpec((64,), lambda b, i: (0,)),
            pl.BlockSpec((64,), lambda b, i: (0,)),
            pl.BlockSpec((64, 64), lambda b, i: (0, 0)),
            pl.BlockSpec((64,), lambda b, i: (0,)),
            pl.BlockSpec((128, 32), lambda b, i: (0, 0)),
            pl.BlockSpec((32,), lambda b, i: (0,)),
            pl.BlockSpec((32, 1), lambda b, i: (0, 0)),
            pl.BlockSpec((1,), lambda b, i: (0,)),
            pl.BlockSpec((256, 128), lambda b, i: (0, 0)),
            pl.BlockSpec((128,), lambda b, i: (0,)),
            pl.BlockSpec((128,), lambda b, i: (0,)),
            pl.BlockSpec((128,), lambda b, i: (0,)),
        ],
        out_specs=[
            pl.BlockSpec((1, _BLK, C), lambda b, i: (b, i, 0)),
            pl.BlockSpec((1, _BLK, 1), lambda b, i: (b, i, 0)),
        ],
        out_shape=[
            jax.ShapeDtypeStruct((B, n, C), jnp.float32),
            jax.ShapeDtypeStruct((B, n, 1), jnp.float32),
        ],
    )(nbrxyz, xyz, feat_diff, features,
      _pad10(geo_w1), geo_b1, geo_g1, geo_be1, geo_w2, geo_b2,
      diff_w1, diff_b1, diff_g1, diff_be1, diff_w2, diff_b2,
      ep_w1, ep_b1, ep_w2, ep_b2, ref_w1, ref_b1, ref_g, ref_be)
    return (refined, edge_prob)


def _pad10(w):
    return jnp.pad(w, ((0, 6), (0, 0)))


# speedup vs baseline: 22.1561x; 1.0695x over previous
import functools

import jax
import jax.numpy as jnp
from jax import lax
from jax.experimental import pallas as pl
from jax.experimental.pallas import tpu as pltpu
from jax.experimental.pallas import tpu_sc as plsc

K = 16
N = 4096
_BLK = 1024
_NEG = -3.0e38

_NW = 32
_PPW = (2 * N) // _NW
_CHP = 16


def _knn_kernel(xyz_blk_ref, xyz_all_ref, idx_ref, pd_ref):
    b = pl.program_id(0)
    xyz_blk = xyz_blk_ref[0]
    xyz_all = xyz_all_ref[0]
    inner = -2.0 * jnp.dot(xyz_blk, xyz_all.T,
                           preferred_element_type=jnp.float32)
    xx_blk = jnp.sum(xyz_blk * xyz_blk, axis=-1, keepdims=True)
    xx_all = jnp.sum(xyz_all * xyz_all, axis=-1, keepdims=True)
    pd_ref[...] = (-xx_blk - inner) - xx_all.T

    iota = jax.lax.broadcasted_iota(jnp.int32, (_BLK, N), 1)
    col16 = jax.lax.broadcasted_iota(jnp.int32, (_BLK, K), 1)

    def body(k, idxs):
        pd = pd_ref[...]
        m = jnp.max(pd, axis=1, keepdims=True)
        am = jnp.min(jnp.where(pd >= m, iota, N), axis=1, keepdims=True)
        pd_ref[...] = jnp.where(iota == am, _NEG, pd)
        return jnp.where(col16 == k, am, idxs)

    idxs = jax.lax.fori_loop(0, K, body, jnp.zeros((_BLK, K), jnp.int32))
    idx_ref[0] = idxs + b * N


def _sc_gather_kernel(feats_hbm, idxflat_hbm, xyzp_hbm,
                      fd_hbm, nbrxyz_hbm,
                      idx_v, rows_v, own_v, out_v, xyzrows_v, xyzpack_v,
                      sem, sem2):
    wid = lax.axis_index("s") * 2 + lax.axis_index("c")

    def chunk_body(g, _):
        base_pt = wid * _PPW + g * _CHP
        pltpu.sync_copy(idxflat_hbm.at[pl.ds(base_pt * K, _CHP * K)], idx_v)
        cp_f = pltpu.async_copy(feats_hbm.at[idx_v], rows_v, sem)
        cp_x = pltpu.async_copy(xyzp_hbm.at[idx_v], xyzrows_v, sem2)
        pltpu.sync_copy(feats_hbm.at[pl.ds(base_pt, _CHP), :], own_v)
        cp_x.wait()

        def pack_body(r, _):
            xyzpack_v[r, :] = xyzrows_v[r, pl.ds(0, 16)]
            return 0

        lax.fori_loop(0, _CHP * K, pack_body, 0)
        pltpu.sync_copy(xyzpack_v,
                        nbrxyz_hbm.at[pl.ds(base_pt * K, _CHP * K), :])
        cp_f.wait()

        def point_body(i, _):
            for j in range(8):
                own_j = own_v[i, pl.ds(16 * j, 16)]
                acc = jnp.abs(own_j - rows_v[i * K, pl.ds(16 * j, 16)])
                for k in range(1, K):
                    acc = acc + jnp.abs(
                        own_j - rows_v[i * K + k, pl.ds(16 * j, 16)])
                out_v[i, pl.ds(16 * j, 16)] = acc * (1.0 / K)
            return 0

        lax.fori_loop(0, _CHP, point_body, 0)
        pltpu.sync_copy(out_v, fd_hbm.at[pl.ds(base_pt, _CHP), :])
        return 0

    lax.fori_loop(0, _PPW // _CHP, chunk_body, 0)


def _sc_gather(feats_flat, idx_flat, xyz_pad):
    mesh = plsc.VectorSubcoreMesh(core_axis_name="c", subcore_axis_name="s")
    run = pl.kernel(
        _sc_gather_kernel,
        mesh=mesh,
        out_type=[
            jax.ShapeDtypeStruct(feats_flat.shape, jnp.float32),
            jax.ShapeDtypeStruct((feats_flat.shape[0] * K, 16), jnp.float32),
        ],
        scratch_types=[
            pltpu.VMEM((_CHP * K,), jnp.int32),
            pltpu.VMEM((_CHP * K, 128), jnp.float32),
            pltpu.VMEM((_CHP, 128), jnp.float32),
            pltpu.VMEM((_CHP, 128), jnp.float32),
            pltpu.VMEM((_CHP * K, 128), jnp.float32),
            pltpu.VMEM((_CHP * K, 16), jnp.float32),
            pltpu.SemaphoreType.DMA,
            pltpu.SemaphoreType.DMA,
        ],
    )
    return run(feats_flat, idx_flat, xyz_pad)


def _geo_tail_kernel(nbr_ref, xyz_blk_ref, fd_ref, f_ref,
                     gw1_ref, gb1_ref, gg1_ref, gbe1_ref, gw2_ref, gb2_ref,
                     dw1_ref, db1_ref, dg1_ref, dbe1_ref, dw2_ref, db2_ref,
                     epw1_ref, epb1_ref, epw2_ref, epb2_ref,
                     rw1_ref, rb1_ref, rg_ref, rbe_ref,
                     refined_ref, prob_ref):
    nb = nbr_ref[0]
    xyz_blk = xyz_blk_ref[0]
    c = [xyz_blk[:, i:i + 1] for i in range(3)]

    dsum = jnp.zeros((_BLK, 1))
    srel = [jnp.zeros((_BLK, 1)) for _ in range(3)]
    srr = [jnp.zeros((_BLK, 1)) for _ in range(6)]
    for k in range(K):
        rel = [nb[:, 16 * k + i:16 * k + i + 1] - c[i] for i in range(3)]
        dsum = dsum + jnp.sqrt(rel[0] * rel[0] + rel[1] * rel[1]
                               + rel[2] * rel[2])
        relb = [r.astype(jnp.bfloat16).astype(jnp.float32) for r in rel]
        srel = [srel[i] + rel[i] for i in range(3)]
        prods = [relb[0] * relb[0], relb[0] * relb[1], relb[0] * relb[2],
                 relb[1] * relb[1], relb[1] * relb[2], relb[2] * relb[2]]
        srr = [srr[i] + prods[i] for i in range(6)]

    inv_k = 1.0 / K
    cidx = {(0, 0): 0, (0, 1): 1, (0, 2): 2, (1, 1): 3, (1, 2): 4, (2, 2): 5}
    a = [[srr[cidx[(min(i, j), max(i, j))]] * inv_k for j in range(3)]
         for i in range(3)]
    mean_rel = [s * inv_k for s in srel]
    mean_dist = dsum * inv_k

    v = [[jnp.full((_BLK, 1), 1.0 if i == j else 0.0) for j in range(3)]
         for i in range(3)]
    for _ in range(15):
        for (p, q) in ((0, 2), (1, 2), (0, 1)):
            apq = a[p][q]
            app = a[p][p]
            aqq = a[q][q]
            safe = jnp.where(apq == 0.0, 1.0, apq)
            tau = (aqq - app) / (2.0 * safe)
            sg = jnp.where(tau >= 0.0, 1.0, -1.0)
            t = sg / (jnp.abs(tau) + jnp.sqrt(1.0 + tau * tau))
            t = jnp.where(apq == 0.0, 0.0, t)
            cc = 1.0 / jnp.sqrt(1.0 + t * t)
            ss = t * cc
            r = 3 - p - q
            new_app = app - t * apq
            new_aqq = aqq + t * apq
            arp = a[r][p]
            arq = a[r][q]
            new_arp = cc * arp - ss * arq
            new_arq = ss * arp + cc * arq
            a[p][p] = new_app
            a[q][q] = new_aqq
            a[p][q] = jnp.zeros_like(apq)
            a[q][p] = a[p][q]
            a[r][p] = new_arp
            a[p][r] = new_arp
            a[r][q] = new_arq
            a[q][r] = new_arq
            for i in range(3):
                vip = v[i][p]
                viq = v[i][q]
                v[i][p] = cc * vip - ss * viq
                v[i][q] = ss * vip + cc * viq

    d0, d1, d2 = a[0][0], a[1][1], a[2][2]
    r0 = (d1 < d0).astype(jnp.int32) + (d2 < d0).astype(jnp.int32)
    r1 = (d0 <= d1).astype(jnp.int32) + (d2 < d1).astype(jnp.int32)
    r2 = (d0 <= d2).astype(jnp.int32) + (d1 <= d2).astype(jnp.int32)

    def pick(slot):
        return [jnp.where(r0 == slot, v[i][0],
                          jnp.where(r1 == slot, v[i][1], v[i][2]))
                for i in range(3)]

    normal = pick(0)
    curv = pick(2)
    geo_feat = jnp.concatenate(
        normal + curv + mean_rel + [mean_dist, jnp.zeros((_BLK, 6))], axis=-1)

    h = jnp.dot(geo_feat, gw1_ref[...],
                preferred_element_type=jnp.float32) + gb1_ref[...]
    mu = jnp.mean(h, axis=-1, keepdims=True)
    var = jnp.mean((h - mu) ** 2, axis=-1, keepdims=True)
    h = (h - mu) / jnp.sqrt(var + 1e-5) * gg1_ref[...] + gbe1_ref[...]
    h = jnp.maximum(h, 0.0)
    geo_enc = jnp.dot(h, gw2_ref[...],
                      preferred_element_type=jnp.float32) + gb2_ref[...]

    fd = fd_ref[0]
    f = f_ref[0]
    d = jnp.dot(fd, dw1_ref[...], preferred_element_type=jnp.float32) + db1_ref[...]
    mu = jnp.mean(d, axis=-1, keepdims=True)
    var = jnp.mean((d - mu) ** 2, axis=-1, keepdims=True)
    d = (d - mu) / jnp.sqrt(var + 1e-5) * dg1_ref[...] + dbe1_ref[...]
    d = jnp.maximum(d, 0.0)
    diff_enc = jnp.dot(d, dw2_ref[...],
                       preferred_element_type=jnp.float32) + db2_ref[...]
    ef = jnp.concatenate([geo_enc, diff_enc], axis=-1)
    e = jnp.maximum(jnp.dot(ef, epw1_ref[...],
                            preferred_element_type=jnp.float32) + epb1_ref[...], 0.0)
    logit = jnp.dot(e, epw2_ref[...],
                    preferred_element_type=jnp.float32) + epb2_ref[...]
    prob = jax.nn.sigmoid(logit)
    enhanced = jnp.concatenate([f, ef * prob], axis=-1)
    r = jnp.dot(enhanced, rw1_ref[...],
                preferred_element_type=jnp.float32) + rb1_ref[...]
    mu = jnp.mean(r, axis=-1, keepdims=True)
    var = jnp.mean((r - mu) ** 2, axis=-1, keepdims=True)
    r = (r - mu) / jnp.sqrt(var + 1e-5) * rg_ref[...] + rbe_ref[...]
    r = jnp.maximum(r, 0.0)
    refined_ref[0] = r + f
    prob_ref[0] = prob


def kernel(xyz, features, geo_w1, geo_b1, geo_g1, geo_be1, geo_w2, geo_b2,
           diff_w1, diff_b1, diff_g1, diff_be1, diff_w2, diff_b2,
           ep_w1, ep_b1, ep_w2, ep_b2, ref_w1, ref_b1, ref_g, ref_be):
    B, n, C = features.shape
    grid = (B, n // _BLK)
    nblk = B * n // _BLK
    flat_idx = pl.pallas_call(
        _knn_kernel,
        grid=grid,
        in_specs=[
            pl.BlockSpec((1, _BLK, 3), lambda b, i: (b, i, 0)),
            pl.BlockSpec((1, N, 3), lambda b, i: (b, 0, 0)),
        ],
        out_specs=pl.BlockSpec((1, _BLK, K),
                               lambda b, i: (b * (n // _BLK) + i, 0, 0)),
        out_shape=jax.ShapeDtypeStruct((nblk, _BLK, K), jnp.int32),
        scratch_shapes=[pltpu.VMEM((_BLK, N), jnp.float32)],
    )(xyz, xyz)

    feats_flat = features.reshape(B * n, C)
    xyz_pad = jnp.pad(xyz.reshape(B * n, 3), ((0, 0), (0, 125)))
    feat_diff, nbrxyz = _sc_gather(feats_flat, flat_idx.reshape(B * n * K),
                                   xyz_pad)
    feat_diff = feat_diff.reshape(B, n, C)
    nbrxyz = nbrxyz.reshape(B, n, K * 16)

    refined, edge_prob = pl.pallas_call(
        _geo_tail_kernel,
        grid=grid,
        in_specs=[
            pl.BlockSpec((1, _BLK, K * 16), lambda b, i: (b, i, 0)),
            pl.BlockSpec((1, _BLK, 3), lambda b, i: (b, i, 0)),
            pl.BlockSpec((1, _BLK, C), lambda b, i: (b, i, 0)),
            pl.BlockSpec((1, _BLK, C), lambda b, i: (b, i, 0)),
            pl.BlockSpec((16, 64), lambda b, i: (0, 0)),
            pl.BlockSpec((64,), lambda b, i: (0,)),
            pl.BlockSpec((64,), lambda b, i: (0,)),
            pl.BlockSpec((64,), lambda b, i: (0,)),
            pl.BlockSpec((64, 64), lambda b, i: (0, 0)),
            pl.BlockSpec((64,), lambda b, i: (0,)),
            pl.BlockSpec((128, 64), lambda b, i: (0, 0)),
            pl.BlockSpec((64,), lambda b, i: (0,)),
            pl.BlockSpec((64,), lambda b, i: (0,)),
            pl.BlockSpec((64,), lambda b, i: (0,)),
            pl.BlockSpec((64, 64), lambda b, i: (0, 0)),
            pl.BlockSpec((64,), lambda b, i: (0,)),
            pl.BlockSpec((128, 32), lambda b, i: (0, 0)),
            pl.BlockSpec((32,), lambda b, i: (0,)),
            pl.BlockSpec((32, 1), lambda b, i: (0, 0)),
            pl.BlockSpec((1,), lambda b, i: (0,)),
            pl.BlockSpec((256, 128), lambda b, i: (0, 0)),
            pl.BlockSpec((128,), lambda b, i: (0,)),
            pl.BlockSpec((128,), lambda b, i: (0,)),
            pl.BlockSpec((128,), lambda b, i: (0,)),
        ],
        out_specs=[
            pl.BlockSpec((1, _BLK, C), lambda b, i: (b, i, 0)),
            pl.BlockSpec((1, _BLK, 1), lambda b, i: (b, i, 0)),
        ],
        out_shape=[
            jax.ShapeDtypeStruct((B, n, C), jnp.float32),
            jax.ShapeDtypeStruct((B, n, 1), jnp.float32),
        ],
    )(nbrxyz, xyz, feat_diff, features,
      _pad10(geo_w1), geo_b1, geo_g1, geo_be1, geo_w2, geo_b2,
      diff_w1, diff_b1, diff_g1, diff_be1, diff_w2, diff_b2,
      ep_w1, ep_b1, ep_w2, ep_b2, ref_w1, ref_b1, ref_g, ref_be)
    return (refined, edge_prob)


def _pad10(w):
    return jnp.pad(w, ((0, 6), (0, 0)))
